# Initial kernel scaffold; baseline (speedup 1.0000x reference)
import functools

import jax
import jax.numpy as jnp
from jax import lax
from jax.experimental import pallas as pl
from jax.experimental.pallas import tpu as pltpu
from jax.experimental.pallas import tpu_sc as plsc

B, V, K, D = 4, 100000, 16, 3
ROW = 16
BD = B * D
C = 128
NW = 32
CPW = 25
CHUNKS = NW * CPW
VPAD = CHUNKS * C
G = (C * K) // C
LANES = 16


def _sc_lap_sq(dT, idx_g, wT_c):
    mesh = plsc.VectorSubcoreMesh(core_axis_name="c", subcore_axis_name="s")

    @functools.partial(
        pl.kernel,
        mesh=mesh,
        out_type=jax.ShapeDtypeStruct((VPAD, ROW), jnp.float32),
        scratch_types=[
            pltpu.VMEM((G, C), jnp.int32),
            pltpu.VMEM((K, C), jnp.float32),
            pltpu.VMEM((C, ROW), jnp.float32),
            pltpu.VMEM((C * K, ROW), jnp.float32),
            pltpu.VMEM((C, ROW), jnp.float32),
            pltpu.SemaphoreType.DMA,
        ],
    )
    def k(dT_hbm, idxg_hbm, wT_hbm, out_hbm, idx_v, w_v, self_v, rows_v,
          out_v, sem):
        wid = lax.axis_index("s") * 2 + lax.axis_index("c")
        lanes = lax.iota(jnp.int32, LANES)

        def chunk_body(ci, carry):
            chunk = wid * CPW + ci
            base = chunk * C
            pltpu.sync_copy(idxg_hbm.at[chunk], idx_v)
            pltpu.sync_copy(wT_hbm.at[chunk], w_v)
            pltpu.sync_copy(dT_hbm.at[pl.ds(base, C)], self_v)
            cps = [
                pltpu.async_copy(dT_hbm.at[idx_v.at[j]],
                                 rows_v.at[pl.ds(j * C, C)], sem)
                for j in range(G)
            ]
            for cp in cps:
                cp.wait()

            def group_body(g, inner):
                rowsel = g * LANES + lanes
                accs = [
                    plsc.load_gather(
                        self_v, [rowsel, jnp.full((LANES,), dd, jnp.int32)])
                    for dd in range(BD)
                ]
                for kk in range(K):
                    wv = w_v[kk, pl.ds(g * LANES, LANES)]
                    rowidx = lanes * K + (g * (LANES * K) + kk)
                    for dd in range(BD):
                        gv = plsc.load_gather(
                            rows_v,
                            [rowidx, jnp.full((LANES,), dd, jnp.int32)])
                        accs[dd] = accs[dd] + wv * gv
                for dd in range(BD):
                    av = accs[dd]
                    plsc.store_scatter(
                        out_v, [rowsel, jnp.full((LANES,), dd, jnp.int32)],
                        av * av)
                return inner

            lax.fori_loop(0, C // LANES, group_body, 0)
            pltpu.sync_copy(out_v, out_hbm.at[pl.ds(base, C)])
            return carry

        lax.fori_loop(0, CPW, chunk_body, 0)

    return k(dT, idx_g, wT_c)


def kernel(out, tgt, neighbor_idxs, neighbor_weights):
    d = (out - tgt).transpose(1, 0, 2).reshape(V, BD)
    dT = jnp.zeros((VPAD, ROW), jnp.float32).at[:V, :BD].set(d)
    idx_pad = jnp.zeros((VPAD, K), jnp.int32).at[:V].set(neighbor_idxs)
    idx_g = idx_pad.reshape(CHUNKS, G, C)
    w_pad = jnp.zeros((VPAD, K), jnp.float32).at[:V].set(neighbor_weights)
    wT_c = w_pad.reshape(CHUNKS, C, K).transpose(0, 2, 1)
    res = _sc_lap_sq(dT, idx_g, wT_c)
    return res[:V, :BD].reshape(V, B, D).transpose(1, 0, 2)


# speedup vs baseline: 30.9933x; 30.9933x over previous
import functools

import jax
import jax.numpy as jnp
from jax import lax
from jax.experimental import pallas as pl
from jax.experimental.pallas import tpu as pltpu
from jax.experimental.pallas import tpu_sc as plsc

B, V, K, D = 4, 100000, 16, 3
ROW = 16
BD = B * D
C = 128
NW = 32
CPW = 25
CHUNKS = NW * CPW
VPAD = CHUNKS * C
G = (C * K) // 128


def _sc_lap_sq(dT, idx_g, w_pad):
    mesh = plsc.VectorSubcoreMesh(core_axis_name="c", subcore_axis_name="s")

    @functools.partial(
        pl.kernel,
        mesh=mesh,
        out_type=jax.ShapeDtypeStruct((VPAD, ROW), jnp.float32),
        compiler_params=pltpu.CompilerParams(use_tc_tiling_on_sc=False),
        scratch_types=[
            pltpu.VMEM((G, 128), jnp.int32),
            pltpu.VMEM((C, K), jnp.float32),
            pltpu.VMEM((C, ROW), jnp.float32),
            pltpu.VMEM((C * K, ROW), jnp.float32),
            pltpu.VMEM((C, ROW), jnp.float32),
            pltpu.SemaphoreType.DMA,
        ],
    )
    def k(dT_hbm, idxg_hbm, w_hbm, out_hbm, idx_v, w_v, self_v, rows_v,
          out_v, sem):
        wid = lax.axis_index("s") * 2 + lax.axis_index("c")

        def chunk_body(ci, carry):
            chunk = wid * CPW + ci
            base = chunk * C
            pltpu.sync_copy(idxg_hbm.at[pl.ds(chunk * G, G)], idx_v)
            pltpu.sync_copy(w_hbm.at[pl.ds(base, C)], w_v)
            pltpu.sync_copy(dT_hbm.at[pl.ds(base, C)], self_v)
            cps = [
                pltpu.async_copy(dT_hbm.at[idx_v.at[j]],
                                 rows_v.at[pl.ds(j * 128, 128)], sem)
                for j in range(G)
            ]
            for cp in cps:
                cp.wait()

            def body(i, c):
                wrow = w_v[i]
                acc = self_v[i]
                for kk in range(K):
                    acc = acc + wrow[kk] * rows_v[i * K + kk]
                out_v[i] = acc * acc
                return c

            lax.fori_loop(0, C, body, 0)
            pltpu.sync_copy(out_v, out_hbm.at[pl.ds(base, C)])
            return carry

        lax.fori_loop(0, CPW, chunk_body, 0)

    return k(dT, idx_g, w_pad)


def kernel(out, tgt, neighbor_idxs, neighbor_weights):
    d = (out - tgt).transpose(1, 0, 2).reshape(V, BD)
    dT = jnp.zeros((VPAD, ROW), jnp.float32).at[:V, :BD].set(d)
    idx_pad = jnp.zeros((VPAD, K), jnp.int32).at[:V].set(neighbor_idxs)
    idx_g = idx_pad.reshape(CHUNKS * G, 128)
    w_pad = jnp.zeros((VPAD, K), jnp.float32).at[:V].set(neighbor_weights)
    res = _sc_lap_sq(dT, idx_g, w_pad)
    return res[:V, :BD].reshape(V, B, D).transpose(1, 0, 2)
